# baseline (speedup 1.0000x reference)
import functools
import math

import jax
import jax.numpy as jnp
from jax import lax
from jax.experimental import pallas as pl
from jax.experimental.pallas import tpu as pltpu
from jax.experimental.pallas import tpu_sc as plsc

_N = 8192
_NUM_BITS = 13
_C = math.exp(-2.0)
_NORM = 1.0 / (1.0 + _C) ** _NUM_BITS
_NS = 16
_PW = _N // _NS
_VPW = _PW // 16

_mesh = plsc.VectorSubcoreMesh(
    core_axis_name="c", subcore_axis_name="s", num_cores=1
)


@functools.partial(
    pl.kernel,
    out_type=(
        jax.ShapeDtypeStruct((_NS, 16), jnp.float32),
        jax.ShapeDtypeStruct((16,), jnp.float32),
    ),
    mesh=_mesh,
    scratch_types=[
        pltpu.VMEM((_PW,), jnp.float32),
        pltpu.VMEM((16,), jnp.int32),
        pltpu.VMEM((1, 16), jnp.float32),
        pltpu.VMEM((_NS, 16), jnp.float32),
        pltpu.VMEM((16,), jnp.float32),
    ],
)
def _attend(q_hbm, mem_hbm, stage_hbm, out_hbm, mem_v, q_v, part_v, red_v, out_v):
    wid = lax.axis_index("s")
    pltpu.sync_copy(q_hbm, q_v)
    pltpu.sync_copy(mem_hbm.at[pl.ds(wid * _PW, _PW)], mem_v)

    qv = q_v[...]
    one = jnp.float32(1.0)
    c = jnp.float32(_C)

    lane = lax.iota(jnp.int32, 16)
    w = jnp.full((16,), jnp.float32(_NORM), dtype=jnp.float32)
    for l in range(4):
        match = ((qv >> l) & 1) == ((lane >> l) & 1)
        w = w * jnp.where(match, one, c)

    x = (jnp.full((16,), 0, dtype=jnp.int32) + wid) ^ (qv >> 9)
    for l in range(4):
        w = w * jnp.where(((x >> l) & 1) == 0, one, c)

    vs = [mem_v[pl.ds(j * 16, 16)] for j in range(_VPW)]
    for l in range(8, 3, -1):
        half = 2 ** (l - 4)
        qb = (qv >> l) & 1
        f0 = jnp.where(qb == 0, one, c)
        f1 = jnp.where(qb == 0, c, one)
        vs = [f0 * vs[j] + f1 * vs[j + half] for j in range(half)]

    part_v[0, :] = vs[0] * w
    pltpu.sync_copy(part_v.at[0], stage_hbm.at[wid])
    plsc.subcore_barrier()

    @pl.when(wid == 0)
    def _():
        pltpu.sync_copy(stage_hbm, red_v)
        acc = red_v[0, :]
        for i in range(1, _NS):
            acc = acc + red_v[i, :]
        lanes = lax.iota(jnp.int32, 16)
        dn = lax.GatherDimensionNumbers(
            offset_dims=(), collapsed_slice_dims=(0,), start_index_map=(0,)
        )
        for sh in (8, 4, 2, 1):
            shuf = lax.gather(
                acc,
                (lanes ^ sh)[:, None],
                dn,
                (1,),
                mode=lax.GatherScatterMode.PROMISE_IN_BOUNDS,
            )
            acc = acc + shuf
        out_v[...] = acc
        pltpu.sync_copy(out_v, out_hbm)


def kernel(query_addr, memory):
    q = jnp.full((16,), query_addr.astype(jnp.int32), dtype=jnp.int32)
    _, out = _attend(q, memory)
    return out[0]


# speedup vs baseline: 1.0411x; 1.0411x over previous
import functools
import math

import jax
import jax.numpy as jnp
from jax import lax
from jax.experimental import pallas as pl
from jax.experimental.pallas import tpu as pltpu
from jax.experimental.pallas import tpu_sc as plsc

_N = 8192
_NUM_BITS = 13
_C = math.exp(-2.0)
_NORM = 1.0 / (1.0 + _C) ** _NUM_BITS
_NS = 16
_PW = _N // _NS
_VPW = _PW // 16

_mesh = plsc.VectorSubcoreMesh(
    core_axis_name="c", subcore_axis_name="s", num_cores=1
)


@functools.partial(
    pl.kernel,
    out_type=jax.ShapeDtypeStruct((16,), jnp.float32),
    mesh=_mesh,
    scratch_types=[
        pltpu.VMEM((_PW,), jnp.float32),
        pltpu.VMEM((16,), jnp.int32),
        pltpu.VMEM((1, 16), jnp.float32),
        pltpu.HBM((_NS, 16), jnp.float32),
        pltpu.VMEM((_NS, 16), jnp.float32),
        pltpu.SemaphoreType.DMA,
    ],
)
def _attend(q_hbm, mem_hbm, out_hbm, mem_v, q_v, part_v, stage_hbm, red_v, sem):
    wid = lax.axis_index("s")
    mem_cp = pltpu.async_copy(mem_hbm.at[pl.ds(wid * _PW, _PW)], mem_v, sem)
    pltpu.sync_copy(q_hbm, q_v)

    qv = q_v[...]
    one = jnp.float32(1.0)
    c = jnp.float32(_C)

    lane = lax.iota(jnp.int32, 16)
    w = jnp.full((16,), jnp.float32(_NORM), dtype=jnp.float32)
    for l in range(4):
        match = ((qv >> l) & 1) == ((lane >> l) & 1)
        w = w * jnp.where(match, one, c)

    x = (jnp.full((16,), 0, dtype=jnp.int32) + wid) ^ (qv >> 9)
    for l in range(4):
        w = w * jnp.where(((x >> l) & 1) == 0, one, c)

    mem_cp.wait()

    vs = [mem_v[pl.ds(j * 16, 16)] for j in range(_VPW)]
    for l in range(8, 3, -1):
        half = 2 ** (l - 4)
        qb = (qv >> l) & 1
        f0 = jnp.where(qb == 0, one, c)
        f1 = jnp.where(qb == 0, c, one)
        vs = [f0 * vs[j] + f1 * vs[j + half] for j in range(half)]

    part_v[0, :] = vs[0] * w
    pltpu.sync_copy(part_v.at[0], stage_hbm.at[wid])
    plsc.subcore_barrier()

    @pl.when(wid == 0)
    def _():
        pltpu.sync_copy(stage_hbm, red_v)
        acc = red_v[0, :]
        for i in range(1, _NS):
            acc = acc + red_v[i, :]
        lanes = lax.iota(jnp.int32, 16)
        dn = lax.GatherDimensionNumbers(
            offset_dims=(), collapsed_slice_dims=(0,), start_index_map=(0,)
        )
        for sh in (8, 4, 2, 1):
            shuf = lax.gather(
                acc,
                (lanes ^ sh)[:, None],
                dn,
                (1,),
                mode=lax.GatherScatterMode.PROMISE_IN_BOUNDS,
            )
            acc = acc + shuf
        part_v[0, :] = acc
        pltpu.sync_copy(part_v.at[0], out_hbm)


def kernel(query_addr, memory):
    q = jnp.full((16,), query_addr.astype(jnp.int32), dtype=jnp.int32)
    out = _attend(q, memory)
    return out[0]
